# baseline (speedup 1.0000x reference)
import functools

import jax
import jax.numpy as jnp
from jax import lax
from jax.experimental import pallas as pl
from jax.experimental.pallas import tpu as pltpu
from jax.experimental.pallas import tpu_sc as plsc

N_NODES = 10000
N_EDGES = 320000
D_FEAT = 128
D_EDGE = 16
D_OUT = 128
DH = D_OUT // 2

NC = 2
NS = 16
NW = NC * NS

NSLAB = 2
ES = N_EDGES // NSLAB
EW = ES // NW
CHUNK = 200
GB = 40
NG = CHUNK // GB
NCHUNK = EW // CHUNK


def _bf16_bits(x):
    u = lax.bitcast_convert_type(x, jnp.uint32)
    return (u + jnp.uint32(0x7FFF) + ((u >> 16) & jnp.uint32(1))) >> 16


def _table_body(x_ref, w_ref, g_ref, wg_ref, b_ref, t_ref):
    c = jnp.dot(g_ref[...], wg_ref[...], preferred_element_type=jnp.float32)
    c = (c + b_ref[...]) * 0.5
    p = jnp.dot(x_ref[...], w_ref[0], preferred_element_type=jnp.float32) + c
    packed = _bf16_bits(p[:, :DH]) | (_bf16_bits(p[:, DH:]) << 16)
    t_ref[0] = lax.bitcast_convert_type(packed, jnp.int32)


def _build_table(x, wrs, g2d, wg, b2d):
    t3 = pl.pallas_call(
        _table_body,
        grid=(2,),
        in_specs=[
            pl.BlockSpec((N_NODES, D_FEAT), lambda j: (0, 0)),
            pl.BlockSpec((1, D_FEAT, D_OUT), lambda j: (j, 0, 0)),
            pl.BlockSpec((1, D_FEAT), lambda j: (0, 0)),
            pl.BlockSpec((D_FEAT, D_OUT), lambda j: (0, 0)),
            pl.BlockSpec((1, D_OUT), lambda j: (0, 0)),
        ],
        out_specs=pl.BlockSpec((1, N_NODES, DH), lambda j: (j, 0, 0)),
        out_shape=jax.ShapeDtypeStruct((2, N_NODES, DH), jnp.int32),
    )(x, wrs, g2d, wg, b2d)
    return t3.reshape(2 * N_NODES, DH)


def _gather_body(slab_base, t_hbm, idx_hbm, out_hbm, idx_v, buf_v, semi, semg,
                 semo):
    wid = lax.axis_index("s") * NC + lax.axis_index("c")
    base = wid * EW

    def idx_copies(cb, b):
        off = slab_base + base + cb * CHUNK
        return [
            pltpu.make_async_copy(
                idx_hbm.at[0, pl.ds(off, CHUNK)], idx_v[b].at[0], semi[b]
            ),
            pltpu.make_async_copy(
                idx_hbm.at[1, pl.ds(off, CHUNK)], idx_v[b].at[1], semi[b]
            ),
        ]

    def out_copies(cb, b):
        off = base + cb * CHUNK
        return [
            pltpu.make_async_copy(
                buf_v[b].at[0], out_hbm.at[pl.ds(off, CHUNK), pl.ds(0, DH)],
                semo[b],
            ),
            pltpu.make_async_copy(
                buf_v[b].at[1], out_hbm.at[pl.ds(off, CHUNK), pl.ds(DH, DH)],
                semo[b],
            ),
        ]

    for c in idx_copies(0, 0):
        c.start()

    def chunk_step(cb, b):
        @pl.when(cb + 1 < NCHUNK)
        def _():
            for c in idx_copies(cb + 1, 1 - b):
                c.start()

        for c in idx_copies(cb, b):
            c.wait()

        @pl.when(cb >= 2)
        def _():
            for c in out_copies(cb - 2, b):
                c.wait()

        gathers = [
            pltpu.async_copy(
                t_hbm.at[idx_v[b].at[h, pl.ds(g * GB, GB)]],
                buf_v[b].at[h].at[pl.ds(g * GB, GB)],
                semg,
            )
            for h in (0, 1)
            for g in range(NG)
        ]
        for c in gathers:
            c.wait()
        for c in out_copies(cb, b):
            c.start()

    def pair_body(ci, carry):
        chunk_step(ci * 2, 0)
        chunk_step(ci * 2 + 1, 1)
        return carry

    lax.fori_loop(0, NCHUNK // 2, pair_body, 0)
    if NCHUNK % 2:
        chunk_step(NCHUNK - 1, 0)
        last_even, last_odd = NCHUNK - 1, NCHUNK - 2
    else:
        last_even, last_odd = NCHUNK - 2, NCHUNK - 1
    for c in out_copies(last_odd, 1):
        c.wait()
    for c in out_copies(last_even, 0):
        c.wait()


@functools.cache
def _make_gather_rows(slab_base):
    return pl.kernel(
        functools.partial(_gather_body, slab_base),
        out_type=jax.ShapeDtypeStruct((ES, D_OUT), jnp.int32),
        mesh=plsc.VectorSubcoreMesh(
            core_axis_name="c",
            subcore_axis_name="s",
            num_cores=NC,
            num_subcores=NS,
        ),
        compiler_params=pltpu.CompilerParams(use_tc_tiling_on_sc=False),
        scratch_types=[
            [pltpu.VMEM((2, CHUNK), jnp.int32)] * 2,
            [pltpu.VMEM((2, CHUNK, DH), jnp.int32)] * 2,
            [pltpu.SemaphoreType.DMA] * 2,
            pltpu.SemaphoreType.DMA,
            [pltpu.SemaphoreType.DMA] * 2,
        ],
    )


BE = 1280
NB = ES // BE


def _unpack_lo_hi(packed_i32):
    u = lax.bitcast_convert_type(packed_i32, jnp.uint32)
    lo = lax.bitcast_convert_type(u << 16, jnp.float32)
    hi = lax.bitcast_convert_type(u & jnp.uint32(0xFFFF0000), jnp.float32)
    return lo, hi


def _fuse_compute(eat_ref, we_ref, g_ref, o_ref):
    acc = lax.dot_general(
        eat_ref[...], we_ref[...],
        (((0,), (0,)), ((), ())),
        preferred_element_type=jnp.float32,
    )
    g = g_ref[...]
    lo_r, hi_r = _unpack_lo_hi(g[:, :DH])
    lo_s, hi_s = _unpack_lo_hi(g[:, DH:])
    o_ref[:, :DH] = jnp.maximum(acc[:, :DH] + lo_r + lo_s, 0.0)
    o_ref[:, DH:] = jnp.maximum(acc[:, DH:] + hi_r + hi_s, 0.0)


def _fuse_body0(eat_ref, we_ref, g_ref, o_ref):
    _fuse_compute(eat_ref, we_ref, g_ref, o_ref)


def _fuse_body1(prev_ref, eat_ref, we_ref, g_ref, o_ref):
    del prev_ref
    _fuse_compute(eat_ref, we_ref, g_ref, o_ref)


def _fuse_slab0(ea_t, we, g_rows):
    return pl.pallas_call(
        _fuse_body0,
        grid=(NB,),
        in_specs=[
            pl.BlockSpec((D_EDGE, BE), lambda i: (0, i)),
            pl.BlockSpec((D_EDGE, D_OUT), lambda i: (0, 0)),
            pl.BlockSpec((BE, D_OUT), lambda i: (i, 0)),
        ],
        out_specs=pl.BlockSpec((BE, D_OUT), lambda i: (i, 0)),
        out_shape=jax.ShapeDtypeStruct((N_EDGES, D_OUT), jnp.float32),
    )(ea_t, we, g_rows)


def _fuse_slab1(prev, ea_t, we, g_rows):
    return pl.pallas_call(
        _fuse_body1,
        grid=(NB,),
        in_specs=[
            pl.BlockSpec(memory_space=pl.ANY),
            pl.BlockSpec((D_EDGE, BE), lambda i: (0, i + NB)),
            pl.BlockSpec((D_EDGE, D_OUT), lambda i: (0, 0)),
            pl.BlockSpec((BE, D_OUT), lambda i: (i, 0)),
        ],
        out_specs=pl.BlockSpec((BE, D_OUT), lambda i: (i + NB, 0)),
        out_shape=jax.ShapeDtypeStruct((N_EDGES, D_OUT), jnp.float32),
        input_output_aliases={0: 0},
    )(prev, ea_t, we, g_rows)


def kernel(edge_attr, x, global_attr, W, b, edge_index):
    we = W[:D_EDGE]
    wrs = W[D_EDGE:D_EDGE + 2 * D_FEAT].reshape(2, D_FEAT, D_OUT)
    wg = W[D_EDGE + 2 * D_FEAT:]
    g2d = global_attr.reshape(1, D_FEAT)
    b2d = b.reshape(1, D_OUT)

    table = _build_table(x, wrs, g2d, wg, b2d)

    idx2 = (
        edge_index + jnp.array([[0], [N_NODES]], dtype=edge_index.dtype)
    ).astype(jnp.int32)

    ea_t = edge_attr.T
    g0 = _make_gather_rows(0)(table, idx2)
    g1 = _make_gather_rows(ES)(table, idx2)
    out = _fuse_slab0(ea_t, we, g0)
    out = _fuse_slab1(out, ea_t, we, g1)
    return out


# speedup vs baseline: 1.3108x; 1.3108x over previous
import functools

import jax
import jax.numpy as jnp
from jax import lax
from jax.experimental import pallas as pl
from jax.experimental.pallas import tpu as pltpu
from jax.experimental.pallas import tpu_sc as plsc

N_NODES = 10000
N_EDGES = 320000
D_FEAT = 128
D_EDGE = 16
D_OUT = 128
DH = D_OUT // 2

NC = 2
NS = 16
NW = NC * NS

NSLAB = 2
ES = N_EDGES // NSLAB
EW = ES // NW
CHUNK = 200
GB = 40
NG = CHUNK // GB
NCHUNK = EW // CHUNK


def _bf16_bits(x):
    u = lax.bitcast_convert_type(x, jnp.uint32)
    return (u + jnp.uint32(0x7FFF) + ((u >> 16) & jnp.uint32(1))) >> 16


def _table_body(x_ref, w_ref, g_ref, wg_ref, b_ref, t_ref):
    c = jnp.dot(g_ref[...], wg_ref[...], preferred_element_type=jnp.float32)
    c = (c + b_ref[...]) * 0.5
    p = jnp.dot(x_ref[...], w_ref[0], preferred_element_type=jnp.float32) + c
    packed = _bf16_bits(p[:, :DH]) | (_bf16_bits(p[:, DH:]) << 16)
    t_ref[0] = lax.bitcast_convert_type(packed, jnp.int32)


def _build_table(x, wrs, g2d, wg, b2d):
    t3 = pl.pallas_call(
        _table_body,
        grid=(2,),
        in_specs=[
            pl.BlockSpec((N_NODES, D_FEAT), lambda j: (0, 0)),
            pl.BlockSpec((1, D_FEAT, D_OUT), lambda j: (j, 0, 0)),
            pl.BlockSpec((1, D_FEAT), lambda j: (0, 0)),
            pl.BlockSpec((D_FEAT, D_OUT), lambda j: (0, 0)),
            pl.BlockSpec((1, D_OUT), lambda j: (0, 0)),
        ],
        out_specs=pl.BlockSpec((1, N_NODES, DH), lambda j: (j, 0, 0)),
        out_shape=jax.ShapeDtypeStruct((2, N_NODES, DH), jnp.int32),
    )(x, wrs, g2d, wg, b2d)
    return t3.reshape(2 * N_NODES, DH)


def _gather_body(slab_base, t_hbm, idx_hbm, out_hbm, idx_v, buf_v, semi, semg,
                 semo):
    wid = lax.axis_index("s") * NC + lax.axis_index("c")
    base = wid * EW

    def idx_copies(cb, b):
        off = slab_base + base + cb * CHUNK
        return [
            pltpu.make_async_copy(
                idx_hbm.at[0, pl.ds(off, CHUNK)], idx_v[b].at[0], semi[b]
            ),
            pltpu.make_async_copy(
                idx_hbm.at[1, pl.ds(off, CHUNK)], idx_v[b].at[1], semi[b]
            ),
        ]

    def out_copies(cb, b):
        off = base + cb * CHUNK
        return [
            pltpu.make_async_copy(
                buf_v[b].at[0], out_hbm.at[pl.ds(off, CHUNK), pl.ds(0, DH)],
                semo[b],
            ),
            pltpu.make_async_copy(
                buf_v[b].at[1], out_hbm.at[pl.ds(off, CHUNK), pl.ds(DH, DH)],
                semo[b],
            ),
        ]

    for c in idx_copies(0, 0):
        c.start()

    def chunk_step(cb, b):
        @pl.when(cb + 1 < NCHUNK)
        def _():
            for c in idx_copies(cb + 1, 1 - b):
                c.start()

        for c in idx_copies(cb, b):
            c.wait()

        @pl.when(cb >= 2)
        def _():
            for c in out_copies(cb - 2, b):
                c.wait()

        gathers = [
            pltpu.async_copy(
                t_hbm.at[idx_v[b].at[h, pl.ds(g * GB, GB)]],
                buf_v[b].at[h].at[pl.ds(g * GB, GB)],
                semg,
            )
            for h in (0, 1)
            for g in range(NG)
        ]
        for c in gathers:
            c.wait()
        for c in out_copies(cb, b):
            c.start()

    def pair_body(ci, carry):
        chunk_step(ci * 2, 0)
        chunk_step(ci * 2 + 1, 1)
        return carry

    lax.fori_loop(0, NCHUNK // 2, pair_body, 0)
    if NCHUNK % 2:
        chunk_step(NCHUNK - 1, 0)
        last_even, last_odd = NCHUNK - 1, NCHUNK - 2
    else:
        last_even, last_odd = NCHUNK - 2, NCHUNK - 1
    for c in out_copies(last_odd, 1):
        c.wait()
    for c in out_copies(last_even, 0):
        c.wait()


@functools.cache
def _make_gather_rows(slab_base):
    return pl.kernel(
        functools.partial(_gather_body, slab_base),
        out_type=jax.ShapeDtypeStruct((ES, D_OUT), jnp.int32),
        mesh=plsc.VectorSubcoreMesh(
            core_axis_name="c",
            subcore_axis_name="s",
            num_cores=NC,
            num_subcores=NS,
        ),
        compiler_params=pltpu.CompilerParams(use_tc_tiling_on_sc=False),
        scratch_types=[
            [pltpu.VMEM((2, CHUNK), jnp.int32)] * 2,
            [pltpu.VMEM((2, CHUNK, DH), jnp.int32)] * 2,
            [pltpu.SemaphoreType.DMA] * 2,
            pltpu.SemaphoreType.DMA,
            [pltpu.SemaphoreType.DMA] * 2,
        ],
    )


BE = 6400
NB = ES // BE


def _unpack_lo_hi(packed_i32):
    u = lax.bitcast_convert_type(packed_i32, jnp.uint32)
    lo = lax.bitcast_convert_type(u << 16, jnp.float32)
    hi = lax.bitcast_convert_type(u & jnp.uint32(0xFFFF0000), jnp.float32)
    return lo, hi


def _fuse_compute(eat_ref, we_ref, g_ref, o_ref):
    acc = lax.dot_general(
        eat_ref[...], we_ref[...],
        (((0,), (0,)), ((), ())),
        preferred_element_type=jnp.float32,
    )
    g = g_ref[...]
    lo_r, hi_r = _unpack_lo_hi(g[:, :DH])
    lo_s, hi_s = _unpack_lo_hi(g[:, DH:])
    o_ref[:, :DH] = jnp.maximum(acc[:, :DH] + lo_r + lo_s, 0.0)
    o_ref[:, DH:] = jnp.maximum(acc[:, DH:] + hi_r + hi_s, 0.0)


def _fuse_body0(eat_ref, we_ref, g_ref, o_ref):
    _fuse_compute(eat_ref, we_ref, g_ref, o_ref)


def _fuse_body1(prev_ref, eat_ref, we_ref, g_ref, o_ref):
    del prev_ref
    _fuse_compute(eat_ref, we_ref, g_ref, o_ref)


def _fuse_slab0(ea_t, we, g_rows):
    return pl.pallas_call(
        _fuse_body0,
        grid=(NB,),
        in_specs=[
            pl.BlockSpec((D_EDGE, BE), lambda i: (0, i)),
            pl.BlockSpec((D_EDGE, D_OUT), lambda i: (0, 0)),
            pl.BlockSpec((BE, D_OUT), lambda i: (i, 0)),
        ],
        out_specs=pl.BlockSpec((BE, D_OUT), lambda i: (i, 0)),
        out_shape=jax.ShapeDtypeStruct((N_EDGES, D_OUT), jnp.float32),
    )(ea_t, we, g_rows)


def _fuse_slab1(prev, ea_t, we, g_rows):
    return pl.pallas_call(
        _fuse_body1,
        grid=(NB,),
        in_specs=[
            pl.BlockSpec(memory_space=pl.ANY),
            pl.BlockSpec((D_EDGE, BE), lambda i: (0, i + NB)),
            pl.BlockSpec((D_EDGE, D_OUT), lambda i: (0, 0)),
            pl.BlockSpec((BE, D_OUT), lambda i: (i, 0)),
        ],
        out_specs=pl.BlockSpec((BE, D_OUT), lambda i: (i + NB, 0)),
        out_shape=jax.ShapeDtypeStruct((N_EDGES, D_OUT), jnp.float32),
        input_output_aliases={0: 0},
    )(prev, ea_t, we, g_rows)


def kernel(edge_attr, x, global_attr, W, b, edge_index):
    we = W[:D_EDGE]
    wrs = W[D_EDGE:D_EDGE + 2 * D_FEAT].reshape(2, D_FEAT, D_OUT)
    wg = W[D_EDGE + 2 * D_FEAT:]
    g2d = global_attr.reshape(1, D_FEAT)
    b2d = b.reshape(1, D_OUT)

    table = _build_table(x, wrs, g2d, wg, b2d)

    idx2 = (
        edge_index + jnp.array([[0], [N_NODES]], dtype=edge_index.dtype)
    ).astype(jnp.int32)

    ea_t = edge_attr.T
    g0 = _make_gather_rows(0)(table, idx2)
    g1 = _make_gather_rows(ES)(table, idx2)
    out = _fuse_slab0(ea_t, we, g0)
    out = _fuse_slab1(out, ea_t, we, g1)
    return out


# speedup vs baseline: 1.4315x; 1.0920x over previous
import functools

import jax
import jax.numpy as jnp
from jax import lax
from jax.experimental import pallas as pl
from jax.experimental.pallas import tpu as pltpu
from jax.experimental.pallas import tpu_sc as plsc

N_NODES = 10000
N_EDGES = 320000
D_FEAT = 128
D_EDGE = 16
D_OUT = 128
DH = D_OUT // 2

NC = 2
NS = 16
NW = NC * NS

NSLAB = 2
ES = N_EDGES // NSLAB
EW = ES // NW
CHUNK = 40
GB = 40
NG = CHUNK // GB
NCHUNK = EW // CHUNK


def _bf16_bits(x):
    u = lax.bitcast_convert_type(x, jnp.uint32)
    return (u + jnp.uint32(0x7FFF) + ((u >> 16) & jnp.uint32(1))) >> 16


def _table_body(x_ref, w_ref, g_ref, wg_ref, b_ref, t_ref):
    c = jnp.dot(g_ref[...], wg_ref[...], preferred_element_type=jnp.float32)
    c = (c + b_ref[...]) * 0.5
    p = jnp.dot(x_ref[...], w_ref[0], preferred_element_type=jnp.float32) + c
    packed = _bf16_bits(p[:, :DH]) | (_bf16_bits(p[:, DH:]) << 16)
    t_ref[0] = lax.bitcast_convert_type(packed, jnp.int32)


def _build_table(x, wrs, g2d, wg, b2d):
    t3 = pl.pallas_call(
        _table_body,
        grid=(2,),
        in_specs=[
            pl.BlockSpec((N_NODES, D_FEAT), lambda j: (0, 0)),
            pl.BlockSpec((1, D_FEAT, D_OUT), lambda j: (j, 0, 0)),
            pl.BlockSpec((1, D_FEAT), lambda j: (0, 0)),
            pl.BlockSpec((D_FEAT, D_OUT), lambda j: (0, 0)),
            pl.BlockSpec((1, D_OUT), lambda j: (0, 0)),
        ],
        out_specs=pl.BlockSpec((1, N_NODES, DH), lambda j: (j, 0, 0)),
        out_shape=jax.ShapeDtypeStruct((2, N_NODES, DH), jnp.int32),
    )(x, wrs, g2d, wg, b2d)
    return t3.reshape(2 * N_NODES, DH)


def _gather_body(slab_base, t_hbm, idx_hbm, out_hbm, t_sh, idx_v, buf_v, semi,
                 semg, semo):
    wid = lax.axis_index("s") * NC + lax.axis_index("c")
    base = wid * EW

    @pl.when(lax.axis_index("s") == 0)
    def _():
        pltpu.sync_copy(t_hbm, t_sh)

    plsc.subcore_barrier()

    def idx_copies(cb, b):
        off = slab_base + base + cb * CHUNK
        return [
            pltpu.make_async_copy(
                idx_hbm.at[0, pl.ds(off, CHUNK)], idx_v[b].at[0], semi[b]
            ),
            pltpu.make_async_copy(
                idx_hbm.at[1, pl.ds(off, CHUNK)], idx_v[b].at[1], semi[b]
            ),
        ]

    def out_copies(cb, b):
        off = base + cb * CHUNK
        return [
            pltpu.make_async_copy(
                buf_v[b].at[0], out_hbm.at[pl.ds(off, CHUNK), pl.ds(0, DH)],
                semo[b],
            ),
            pltpu.make_async_copy(
                buf_v[b].at[1], out_hbm.at[pl.ds(off, CHUNK), pl.ds(DH, DH)],
                semo[b],
            ),
        ]

    for c in idx_copies(0, 0):
        c.start()

    def chunk_step(cb, b):
        @pl.when(cb + 1 < NCHUNK)
        def _():
            for c in idx_copies(cb + 1, 1 - b):
                c.start()

        for c in idx_copies(cb, b):
            c.wait()

        @pl.when(cb >= 2)
        def _():
            for c in out_copies(cb - 2, b):
                c.wait()

        gathers = [
            pltpu.async_copy(
                t_sh.at[idx_v[b].at[h, pl.ds(g * GB, GB)]],
                buf_v[b].at[h].at[pl.ds(g * GB, GB)],
                semg,
            )
            for h in (0, 1)
            for g in range(NG)
        ]
        for c in gathers:
            c.wait()
        for c in out_copies(cb, b):
            c.start()

    def pair_body(ci, carry):
        chunk_step(ci * 2, 0)
        chunk_step(ci * 2 + 1, 1)
        return carry

    lax.fori_loop(0, NCHUNK // 2, pair_body, 0)
    if NCHUNK % 2:
        chunk_step(NCHUNK - 1, 0)
        last_even, last_odd = NCHUNK - 1, NCHUNK - 2
    else:
        last_even, last_odd = NCHUNK - 2, NCHUNK - 1
    for c in out_copies(last_odd, 1):
        c.wait()
    for c in out_copies(last_even, 0):
        c.wait()


@functools.cache
def _make_gather_rows(slab_base):
    return pl.kernel(
        functools.partial(_gather_body, slab_base),
        out_type=jax.ShapeDtypeStruct((ES, D_OUT), jnp.int32),
        mesh=plsc.VectorSubcoreMesh(
            core_axis_name="c",
            subcore_axis_name="s",
            num_cores=NC,
            num_subcores=NS,
        ),
        compiler_params=pltpu.CompilerParams(use_tc_tiling_on_sc=False),
        scratch_types=[
            pltpu.VMEM_SHARED((2 * N_NODES, DH), jnp.int32),
            [pltpu.VMEM((2, CHUNK), jnp.int32)] * 2,
            [pltpu.VMEM((2, CHUNK, DH), jnp.int32)] * 2,
            [pltpu.SemaphoreType.DMA] * 2,
            pltpu.SemaphoreType.DMA,
            [pltpu.SemaphoreType.DMA] * 2,
        ],
    )


BE = 6400
NB = ES // BE


def _unpack_lo_hi(packed_i32):
    u = lax.bitcast_convert_type(packed_i32, jnp.uint32)
    lo = lax.bitcast_convert_type(u << 16, jnp.float32)
    hi = lax.bitcast_convert_type(u & jnp.uint32(0xFFFF0000), jnp.float32)
    return lo, hi


def _fuse_compute(eat_ref, we_ref, g_ref, o_ref):
    acc = lax.dot_general(
        eat_ref[...], we_ref[...],
        (((0,), (0,)), ((), ())),
        preferred_element_type=jnp.float32,
    )
    g = g_ref[...]
    lo_r, hi_r = _unpack_lo_hi(g[:, :DH])
    lo_s, hi_s = _unpack_lo_hi(g[:, DH:])
    o_ref[:, :DH] = jnp.maximum(acc[:, :DH] + lo_r + lo_s, 0.0)
    o_ref[:, DH:] = jnp.maximum(acc[:, DH:] + hi_r + hi_s, 0.0)


def _fuse_body0(eat_ref, we_ref, g_ref, o_ref):
    _fuse_compute(eat_ref, we_ref, g_ref, o_ref)


def _fuse_body1(prev_ref, eat_ref, we_ref, g_ref, o_ref):
    del prev_ref
    _fuse_compute(eat_ref, we_ref, g_ref, o_ref)


def _fuse_slab0(ea_t, we, g_rows):
    return pl.pallas_call(
        _fuse_body0,
        grid=(NB,),
        in_specs=[
            pl.BlockSpec((D_EDGE, BE), lambda i: (0, i)),
            pl.BlockSpec((D_EDGE, D_OUT), lambda i: (0, 0)),
            pl.BlockSpec((BE, D_OUT), lambda i: (i, 0)),
        ],
        out_specs=pl.BlockSpec((BE, D_OUT), lambda i: (i, 0)),
        out_shape=jax.ShapeDtypeStruct((N_EDGES, D_OUT), jnp.float32),
    )(ea_t, we, g_rows)


def _fuse_slab1(prev, ea_t, we, g_rows):
    return pl.pallas_call(
        _fuse_body1,
        grid=(NB,),
        in_specs=[
            pl.BlockSpec(memory_space=pl.ANY),
            pl.BlockSpec((D_EDGE, BE), lambda i: (0, i + NB)),
            pl.BlockSpec((D_EDGE, D_OUT), lambda i: (0, 0)),
            pl.BlockSpec((BE, D_OUT), lambda i: (i, 0)),
        ],
        out_specs=pl.BlockSpec((BE, D_OUT), lambda i: (i + NB, 0)),
        out_shape=jax.ShapeDtypeStruct((N_EDGES, D_OUT), jnp.float32),
        input_output_aliases={0: 0},
    )(prev, ea_t, we, g_rows)


def kernel(edge_attr, x, global_attr, W, b, edge_index):
    we = W[:D_EDGE]
    wrs = W[D_EDGE:D_EDGE + 2 * D_FEAT].reshape(2, D_FEAT, D_OUT)
    wg = W[D_EDGE + 2 * D_FEAT:]
    g2d = global_attr.reshape(1, D_FEAT)
    b2d = b.reshape(1, D_OUT)

    table = _build_table(x, wrs, g2d, wg, b2d)

    idx2 = (
        edge_index + jnp.array([[0], [N_NODES]], dtype=edge_index.dtype)
    ).astype(jnp.int32)

    ea_t = edge_attr.T
    g0 = _make_gather_rows(0)(table, idx2)
    g1 = _make_gather_rows(ES)(table, idx2)
    out = _fuse_slab0(ea_t, we, g0)
    out = _fuse_slab1(out, ea_t, we, g1)
    return out
